# baseline (speedup 1.0000x reference)
import functools

import jax
import jax.numpy as jnp
from jax import lax
from jax.experimental import pallas as pl
from jax.experimental.pallas import tpu as pltpu
from jax.experimental.pallas import tpu_sc as plsc

_ED = 16
_IDX_CHUNK = 128


def _repack_body(ta, tb, oa, ob):
    for ref, oref in ((ta, oa), (tb, ob)):
        v3 = ref[...].reshape(oref.shape[0], 8, _ED)
        oref[...] = jnp.concatenate([v3[:, g, :] for g in range(8)], axis=1)


@functools.cache
def _repack(V: int):
    rows = ((V // 8) + 1599) // 1600 * 1600
    blk = rows // 8
    return pl.pallas_call(
        _repack_body,
        grid=(8,),
        in_specs=[
            pl.BlockSpec((blk * 8, _ED), lambda i: (i, 0)),
            pl.BlockSpec((blk * 8, _ED), lambda i: (i, 0)),
        ],
        out_specs=[
            pl.BlockSpec((blk, 128), lambda i: (i, 0)),
            pl.BlockSpec((blk, 128), lambda i: (i, 0)),
        ],
        out_shape=[
            jax.ShapeDtypeStruct((rows, 128), jnp.float32),
            jax.ShapeDtypeStruct((rows, 128), jnp.float32),
        ],
    )


@functools.cache
def _sc_group_gather(B: int, V: int):
    info = plsc.get_sparse_core_info()
    nc, ns = info.num_cores, info.num_subcores
    nw = nc * ns
    bpw = B // nw
    half = bpw // 2
    n_chunks = half // _IDX_CHUNK
    assert B % nw == 0 and half % _IDX_CHUNK == 0
    mesh = plsc.VectorSubcoreMesh(core_axis_name="c", subcore_axis_name="s")

    @functools.partial(
        pl.kernel,
        out_type=(
            jax.ShapeDtypeStruct((B, 128), jnp.float32),
            jax.ShapeDtypeStruct((B, 128), jnp.float32),
        ),
        mesh=mesh,
        compiler_params=pltpu.CompilerParams(use_tc_tiling_on_sc=False),
        scratch_types=[
            pltpu.VMEM((bpw,), jnp.int32),
            pltpu.VMEM((bpw,), jnp.int32),
            pltpu.VMEM((half, 128), jnp.float32),
            pltpu.VMEM((half, 128), jnp.float32),
            pltpu.SemaphoreType.DMA,
            pltpu.SemaphoreType.DMA,
        ],
    )
    def gather2(g_a_hbm, g_b_hbm, tab_a, tab_b, out_a, out_b,
                idx_a, idx_b, buf_a, buf_b, sem_a, sem_b):
        wid = lax.axis_index("s") * nc + lax.axis_index("c")
        base = wid * bpw
        ca = pltpu.async_copy(g_a_hbm.at[pl.ds(base, bpw)], idx_a, sem_a)
        cb = pltpu.async_copy(g_b_hbm.at[pl.ds(base, bpw)], idx_b, sem_b)
        ca.wait()
        cb.wait()
        for h in range(2):
            copies = []
            for j in range(n_chunks):
                lo = h * half + j * _IDX_CHUNK
                copies.append(pltpu.async_copy(
                    tab_a.at[idx_a.at[pl.ds(lo, _IDX_CHUNK)]],
                    buf_a.at[pl.ds(j * _IDX_CHUNK, _IDX_CHUNK)], sem_a))
                copies.append(pltpu.async_copy(
                    tab_b.at[idx_b.at[pl.ds(lo, _IDX_CHUNK)]],
                    buf_b.at[pl.ds(j * _IDX_CHUNK, _IDX_CHUNK)], sem_b))
            for c in copies:
                c.wait()
            wa = pltpu.async_copy(buf_a, out_a.at[pl.ds(base + h * half, half)], sem_a)
            wb = pltpu.async_copy(buf_b, out_b.at[pl.ds(base + h * half, half)], sem_b)
            wa.wait()
            wb.wait()

    return gather2


def _mlp_body(ga, gb, xr, w1a, w1b, w1n, b1, w2, b2, w3, b3, wo, bo, out):
    ids_a = xr[:, 0:1].astype(jnp.int32)
    ids_b = xr[:, 1:2].astype(jnp.int32)
    sa = jnp.bitwise_and(ids_a, 7)
    sb = jnp.bitwise_and(ids_b, 7)
    ea = jnp.zeros((xr.shape[0], _ED), jnp.float32)
    eb = jnp.zeros((xr.shape[0], _ED), jnp.float32)
    for k in range(8):
        ea = ea + jnp.where(sa == k, ga[:, _ED * k:_ED * (k + 1)], 0.0)
        eb = eb + jnp.where(sb == k, gb[:, _ED * k:_ED * (k + 1)], 0.0)
    num = xr[:, 2:]
    h = jnp.dot(ea, w1a[...], preferred_element_type=jnp.float32)
    h = h + jnp.dot(eb, w1b[...], preferred_element_type=jnp.float32)
    h = h + jnp.dot(num, w1n[...], preferred_element_type=jnp.float32)
    h = jnp.maximum(h + b1[...], 0.0)
    h = jnp.maximum(
        jnp.dot(h, w2[...], preferred_element_type=jnp.float32) + b2[...], 0.0)
    h = jnp.maximum(
        jnp.dot(h, w3[...], preferred_element_type=jnp.float32) + b3[...], 0.0)
    z = jnp.sum(h * wo[...], axis=1, keepdims=True) + bo[...]
    out[...] = 1.0 / (1.0 + jnp.exp(-z))


@functools.cache
def _mlp_call(B: int, F: int, blk: int):
    full = lambda shape: pl.BlockSpec(shape, lambda i: (0, 0))
    return pl.pallas_call(
        _mlp_body,
        grid=(B // blk,),
        in_specs=[
            pl.BlockSpec((blk, 128), lambda i: (i, 0)),
            pl.BlockSpec((blk, 128), lambda i: (i, 0)),
            pl.BlockSpec((blk, F + 2), lambda i: (i, 0)),
            full((_ED, 64)),
            full((_ED, 64)),
            full((F, 64)),
            full((1, 64)),
            full((64, 32)),
            full((1, 32)),
            full((32, 16)),
            full((1, 16)),
            full((1, 16)),
            full((1, 1)),
        ],
        out_specs=pl.BlockSpec((blk, 1), lambda i: (i, 0)),
        out_shape=jax.ShapeDtypeStruct((B, 1), jnp.float32),
    )


def kernel(x, emb_a, emb_b, W1, b1, W2, b2, W3, b3, Wo, bo):
    B, C = x.shape
    F = C - 2
    V = emb_a.shape[0]
    g_a = x[:, 0].astype(jnp.int32) >> 3
    g_b = x[:, 1].astype(jnp.int32) >> 3
    ta8, tb8 = _repack(V)(emb_a, emb_b)
    ga, gb = _sc_group_gather(B, V)(g_a, g_b, ta8, tb8)
    W1T = W1.T
    out = _mlp_call(B, F, 2048)(
        ga, gb, x,
        W1T[:_ED], W1T[_ED:2 * _ED], W1T[2 * _ED:],
        b1.reshape(1, 64), W2.T, b2.reshape(1, 32), W3.T, b3.reshape(1, 16),
        Wo, bo.reshape(1, 1))
    return out


# speedup vs baseline: 2.0979x; 2.0979x over previous
import functools

import jax
import jax.numpy as jnp
from jax import lax
from jax.experimental import pallas as pl
from jax.experimental.pallas import tpu as pltpu
from jax.experimental.pallas import tpu_sc as plsc

_ED = 16
_IDX_CHUNK = 128
_LANES = 3200


def _iota16():
    return lax.iota(jnp.int32, 16)


@functools.cache
def _sc_repack(V: int):
    info = plsc.get_sparse_core_info()
    nc, ns = info.num_cores, info.num_subcores
    nw = nc * ns
    q_all = V // 8
    tail_lo = (nw - 1) * _LANES
    tail_w = (V - tail_lo) // 128 * 128
    assert 0 < tail_w <= _LANES
    mesh = plsc.VectorSubcoreMesh(core_axis_name="c", subcore_axis_name="s")

    @functools.partial(
        pl.kernel,
        out_type=(
            jax.ShapeDtypeStruct((q_all * 128,), jnp.float32),
            jax.ShapeDtypeStruct((q_all * 128,), jnp.float32),
        ),
        mesh=mesh,
        compiler_params=pltpu.CompilerParams(needs_layout_passes=False),
        scratch_types=[
            pltpu.VMEM((16, _LANES), jnp.float32),
            pltpu.VMEM((16 * _LANES,), jnp.float32),
        ],
    )
    def repack(taT, tbT, pa, pb, buf, stage):
        wid = lax.axis_index("s") * nc + lax.axis_index("c")
        iota = _iota16()
        sbase = (iota // 8) * 128 + 16 * (iota % 8)

        def do_slice(src, dst, lo, width):
            nq = width // 8
            lo = pl.multiple_of(lo, 128)
            pltpu.sync_copy(src.at[:, pl.ds(lo, width)],
                            buf.at[:, pl.ds(0, width)])

            def body(t2, _):
                base_flat = 256 * t2 + sbase
                for f in range(16):
                    vals = buf[f, pl.ds(16 * t2, 16)]
                    plsc.store_scatter(stage, [base_flat + f], vals)
                return _

            lax.fori_loop(0, nq // 2, body, 0)
            pltpu.sync_copy(stage.at[pl.ds(0, nq * 128)],
                            dst.at[pl.ds(pl.multiple_of(lo * 16, 2048), nq * 128)])

        @pl.when(wid < nw - 1)
        def _():
            lo = wid * _LANES
            do_slice(taT, pa, lo, _LANES)
            do_slice(tbT, pb, lo, _LANES)

        @pl.when(wid == nw - 1)
        def _():
            do_slice(taT, pa, tail_lo, tail_w)
            do_slice(tbT, pb, tail_lo, tail_w)

    return repack


@functools.cache
def _sc_gather_extract(B: int, V: int):
    info = plsc.get_sparse_core_info()
    nc, ns = info.num_cores, info.num_subcores
    nw = nc * ns
    bpw = B // nw
    half = bpw // 2
    n_chunks = half // _IDX_CHUNK
    assert B % nw == 0 and half % _IDX_CHUNK == 0
    mesh = plsc.VectorSubcoreMesh(core_axis_name="c", subcore_axis_name="s")

    @functools.partial(
        pl.kernel,
        out_type=(
            jax.ShapeDtypeStruct((B, _ED), jnp.float32),
            jax.ShapeDtypeStruct((B, _ED), jnp.float32),
        ),
        mesh=mesh,
        compiler_params=pltpu.CompilerParams(use_tc_tiling_on_sc=False,
                                             needs_layout_passes=False),
        scratch_types=[
            pltpu.VMEM((bpw,), jnp.int32),
            pltpu.VMEM((bpw,), jnp.int32),
            pltpu.VMEM((bpw,), jnp.int32),
            pltpu.VMEM((bpw,), jnp.int32),
            pltpu.VMEM((half, 128), jnp.float32),
            pltpu.VMEM((half, 128), jnp.float32),
            pltpu.VMEM((bpw, _ED), jnp.float32),
            pltpu.VMEM((bpw, _ED), jnp.float32),
            pltpu.SemaphoreType.DMA,
            pltpu.SemaphoreType.DMA,
        ],
    )
    def gather2(ids_a_hbm, ids_b_hbm, pa, pb, ea, eb,
                ids_a, ids_b, g_a, g_b, buf_a, buf_b, out_a, out_b,
                sem_a, sem_b):
        wid = lax.axis_index("s") * nc + lax.axis_index("c")
        base = pl.multiple_of(wid * bpw, bpw)
        ca = pltpu.async_copy(ids_a_hbm.at[pl.ds(base, bpw)], ids_a, sem_a)
        cb = pltpu.async_copy(ids_b_hbm.at[pl.ds(base, bpw)], ids_b, sem_b)
        ca.wait()
        cb.wait()
        iota = _iota16()

        def shift_body(t, _):
            sl = pl.ds(16 * t, 16)
            g_a[sl] = lax.shift_right_logical(ids_a[sl], 3)
            g_b[sl] = lax.shift_right_logical(ids_b[sl], 3)
            return _

        lax.fori_loop(0, bpw // 16, shift_body, 0)

        for h in range(2):
            copies = []
            for j in range(n_chunks):
                lo = h * half + j * _IDX_CHUNK
                copies.append(pltpu.async_copy(
                    pa.at[g_a.at[pl.ds(lo, _IDX_CHUNK)]],
                    buf_a.at[pl.ds(j * _IDX_CHUNK, _IDX_CHUNK)], sem_a))
                copies.append(pltpu.async_copy(
                    pb.at[g_b.at[pl.ds(lo, _IDX_CHUNK)]],
                    buf_b.at[pl.ds(j * _IDX_CHUNK, _IDX_CHUNK)], sem_b))
            for c in copies:
                c.wait()

            def extract_body(t, _):
                row_vec = 16 * t + iota
                sa = lax.bitwise_and(ids_a[pl.ds(h * half + 16 * t, 16)], 7)
                sb = lax.bitwise_and(ids_b[pl.ds(h * half + 16 * t, 16)], 7)
                orow = h * half + 16 * t + iota
                for f in range(16):
                    va = plsc.load_gather(buf_a, [row_vec, sa * 16 + f])
                    plsc.store_scatter(out_a, [orow, jnp.full((16,), f, jnp.int32)], va)
                    vb = plsc.load_gather(buf_b, [row_vec, sb * 16 + f])
                    plsc.store_scatter(out_b, [orow, jnp.full((16,), f, jnp.int32)], vb)
                return _

            lax.fori_loop(0, half // 16, extract_body, 0)

        wa = pltpu.async_copy(out_a, ea.at[pl.ds(base, bpw)], sem_a)
        wb = pltpu.async_copy(out_b, eb.at[pl.ds(base, bpw)], sem_b)
        wa.wait()
        wb.wait()

    return gather2


def _mlp_body(ea, eb, num, w1a, w1b, w1n, b1, w2, b2, w3, b3, wo, bo, out):
    h = jnp.dot(ea[...], w1a[...], preferred_element_type=jnp.float32)
    h = h + jnp.dot(eb[...], w1b[...], preferred_element_type=jnp.float32)
    h = h + jnp.dot(num[...], w1n[...], preferred_element_type=jnp.float32)
    h = jnp.maximum(h + b1[...], 0.0)
    h = jnp.maximum(
        jnp.dot(h, w2[...], preferred_element_type=jnp.float32) + b2[...], 0.0)
    h = jnp.maximum(
        jnp.dot(h, w3[...], preferred_element_type=jnp.float32) + b3[...], 0.0)
    z = jnp.sum(h * wo[...], axis=1, keepdims=True) + bo[...]
    out[...] = 1.0 / (1.0 + jnp.exp(-z))


@functools.cache
def _mlp_call(B: int, F: int, blk: int):
    full = lambda shape: pl.BlockSpec(shape, lambda i: (0, 0))
    return pl.pallas_call(
        _mlp_body,
        grid=(B // blk,),
        in_specs=[
            pl.BlockSpec((blk, _ED), lambda i: (i, 0)),
            pl.BlockSpec((blk, _ED), lambda i: (i, 0)),
            pl.BlockSpec((blk, F), lambda i: (i, 0)),
            full((_ED, 64)),
            full((_ED, 64)),
            full((F, 64)),
            full((1, 64)),
            full((64, 32)),
            full((1, 32)),
            full((32, 16)),
            full((1, 16)),
            full((1, 16)),
            full((1, 1)),
        ],
        out_specs=pl.BlockSpec((blk, 1), lambda i: (i, 0)),
        out_shape=jax.ShapeDtypeStruct((B, 1), jnp.float32),
    )


def kernel(x, emb_a, emb_b, W1, b1, W2, b2, W3, b3, Wo, bo):
    B, C = x.shape
    F = C - 2
    V = emb_a.shape[0]
    ids_a = x[:, 0].astype(jnp.int32)
    ids_b = x[:, 1].astype(jnp.int32)
    numeric = x[:, 2:]
    pa, pb = _sc_repack(V)(emb_a.T, emb_b.T)
    covered = V // 128 * 128
    if covered < V:
        pa = pa.at[covered * _ED:].set(emb_a[covered:].reshape(-1))
        pb = pb.at[covered * _ED:].set(emb_b[covered:].reshape(-1))
    pa = pa.reshape(V // 8, 128)
    pb = pb.reshape(V // 8, 128)
    ea, eb = _sc_gather_extract(B, V)(ids_a, ids_b, pa, pb)
    W1T = W1.T
    out = _mlp_call(B, F, 2048)(
        ea, eb, numeric,
        W1T[:_ED], W1T[_ED:2 * _ED], W1T[2 * _ED:],
        b1.reshape(1, 64), W2.T, b2.reshape(1, 32), W3.T, b3.reshape(1, 16),
        Wo, bo.reshape(1, 1))
    return out


# speedup vs baseline: 2.5376x; 1.2096x over previous
import functools

import jax
import jax.numpy as jnp
from jax import lax
from jax.experimental import pallas as pl
from jax.experimental.pallas import tpu as pltpu
from jax.experimental.pallas import tpu_sc as plsc

_ED = 16
_IDX_CHUNK = 128
_LANES = 3200
_CW = 640


def _iota16():
    return lax.iota(jnp.int32, 16)


@functools.cache
def _sc_repack(V: int):
    info = plsc.get_sparse_core_info()
    nc, ns = info.num_cores, info.num_subcores
    nw = nc * ns
    tail_lo = (nw - 1) * _LANES
    tail_w = (V - tail_lo) // 128 * 128
    tail_cw = 384
    assert tail_w % tail_cw == 0 and _LANES % _CW == 0
    mesh = plsc.VectorSubcoreMesh(core_axis_name="c", subcore_axis_name="s")

    @functools.partial(
        pl.kernel,
        out_type=(
            jax.ShapeDtypeStruct((V // 8 * 128,), jnp.float32),
            jax.ShapeDtypeStruct((V // 8 * 128,), jnp.float32),
        ),
        mesh=mesh,
        compiler_params=pltpu.CompilerParams(needs_layout_passes=False),
        scratch_types=[
            pltpu.VMEM((16, _CW), jnp.float32),
            pltpu.VMEM((16, _CW), jnp.float32),
            pltpu.VMEM((16 * _CW,), jnp.float32),
            pltpu.VMEM((16 * _CW,), jnp.float32),
            pltpu.SemaphoreType.DMA,
            pltpu.SemaphoreType.DMA,
            pltpu.SemaphoreType.DMA,
            pltpu.SemaphoreType.DMA,
        ],
    )
    def repack(taT, tbT, pa, pb, buf0, buf1, st0, st1,
               sin0, sin1, sout0, sout1):
        wid = lax.axis_index("s") * nc + lax.axis_index("c")
        iota = _iota16()
        sbase = (iota // 8) * 128 + 16 * (iota % 8)
        bufs = (buf0, buf1)
        stages = (st0, st1)
        sins = (sin0, sin1)
        souts = (sout0, sout1)

        def pack_chunk(buf, stage, cw):
            def body(t2, _):
                base_flat = 256 * t2 + sbase
                for f in range(16):
                    vals = buf[f, pl.ds(16 * t2, 16)]
                    plsc.store_scatter(stage, [base_flat + f], vals)
                return _

            lax.fori_loop(0, cw // 16, body, 0)

        def run(chunks, cw):
            n = len(chunks)
            infl_in = [None, None]
            infl_out = [None, None]

            def start_in(i):
                src, _, lo = chunks[i]
                p = i % 2
                infl_in[p] = pltpu.async_copy(
                    src.at[:, pl.ds(pl.multiple_of(lo, 128), cw)],
                    bufs[p].at[:, pl.ds(0, cw)], sins[p])

            start_in(0)
            for i in range(n):
                p = i % 2
                infl_in[p].wait()
                if i + 1 < n:
                    start_in(i + 1)
                if infl_out[p] is not None:
                    infl_out[p].wait()
                pack_chunk(bufs[p], stages[p], cw)
                _, dst, lo = chunks[i]
                infl_out[p] = pltpu.async_copy(
                    stages[p].at[pl.ds(0, cw * 16)],
                    dst.at[pl.ds(pl.multiple_of(lo * 16, 2048), cw * 16)],
                    souts[p])
            for c in infl_out:
                if c is not None:
                    c.wait()

        @pl.when(wid < nw - 1)
        def _():
            lo = pl.multiple_of(wid * _LANES, _LANES)
            chunks = []
            for tab, dst in ((taT, pa), (tbT, pb)):
                for k in range(_LANES // _CW):
                    chunks.append((tab, dst, lo + k * _CW))
            run(chunks, _CW)

        @pl.when(wid == nw - 1)
        def _():
            chunks = []
            for tab, dst in ((taT, pa), (tbT, pb)):
                for k in range(tail_w // tail_cw):
                    chunks.append((tab, dst, tail_lo + k * tail_cw))
            run(chunks, tail_cw)

    return repack


@functools.cache
def _sc_gather2(B: int, V: int):
    info = plsc.get_sparse_core_info()
    nc, ns = info.num_cores, info.num_subcores
    nw = nc * ns
    bpw = B // nw
    n_chunks = bpw // _IDX_CHUNK
    assert bpw % _IDX_CHUNK == 0 and B % nw == 0
    mesh = plsc.VectorSubcoreMesh(core_axis_name="c", subcore_axis_name="s")

    @functools.partial(
        pl.kernel,
        out_type=(
            jax.ShapeDtypeStruct((B, _ED), jnp.float32),
            jax.ShapeDtypeStruct((B, _ED), jnp.float32),
        ),
        mesh=mesh,
        compiler_params=pltpu.CompilerParams(use_tc_tiling_on_sc=False),
        scratch_types=[
            pltpu.VMEM((bpw,), jnp.int32),
            pltpu.VMEM((bpw,), jnp.int32),
            pltpu.VMEM((bpw, _ED), jnp.float32),
            pltpu.VMEM((bpw, _ED), jnp.float32),
            pltpu.SemaphoreType.DMA,
            pltpu.SemaphoreType.DMA,
        ],
    )
    def gather2(ids_a_hbm, ids_b_hbm, tab_a, tab_b, out_a, out_b,
                idx_a, idx_b, rows_a, rows_b, sem_a, sem_b):
        wid = lax.axis_index("s") * nc + lax.axis_index("c")
        base = pl.multiple_of(wid * bpw, bpw)
        ca = pltpu.async_copy(ids_a_hbm.at[pl.ds(base, bpw)], idx_a, sem_a)
        cb = pltpu.async_copy(ids_b_hbm.at[pl.ds(base, bpw)], idx_b, sem_b)
        ca.wait()
        cb.wait()
        copies = []
        for j in range(n_chunks):
            copies.append(pltpu.async_copy(
                tab_a.at[idx_a.at[pl.ds(j * _IDX_CHUNK, _IDX_CHUNK)]],
                rows_a.at[pl.ds(j * _IDX_CHUNK, _IDX_CHUNK)], sem_a))
            copies.append(pltpu.async_copy(
                tab_b.at[idx_b.at[pl.ds(j * _IDX_CHUNK, _IDX_CHUNK)]],
                rows_b.at[pl.ds(j * _IDX_CHUNK, _IDX_CHUNK)], sem_b))
        for c in copies:
            c.wait()
        wa = pltpu.async_copy(rows_a, out_a.at[pl.ds(base, bpw)], sem_a)
        wb = pltpu.async_copy(rows_b, out_b.at[pl.ds(base, bpw)], sem_b)
        wa.wait()
        wb.wait()

    return gather2


def _mlp_body(ea, eb, num, w1a, w1b, w1n, b1, w2, b2, w3, b3, wo, bo, out):
    h = jnp.dot(ea[...], w1a[...], preferred_element_type=jnp.float32)
    h = h + jnp.dot(eb[...], w1b[...], preferred_element_type=jnp.float32)
    h = h + jnp.dot(num[...], w1n[...], preferred_element_type=jnp.float32)
    h = jnp.maximum(h + b1[...], 0.0)
    h = jnp.maximum(
        jnp.dot(h, w2[...], preferred_element_type=jnp.float32) + b2[...], 0.0)
    h = jnp.maximum(
        jnp.dot(h, w3[...], preferred_element_type=jnp.float32) + b3[...], 0.0)
    z = jnp.sum(h * wo[...], axis=1, keepdims=True) + bo[...]
    out[...] = 1.0 / (1.0 + jnp.exp(-z))


@functools.cache
def _mlp_call(B: int, F: int, blk: int):
    full = lambda shape: pl.BlockSpec(shape, lambda i: (0, 0))
    return pl.pallas_call(
        _mlp_body,
        grid=(B // blk,),
        in_specs=[
            pl.BlockSpec((blk, _ED), lambda i: (i, 0)),
            pl.BlockSpec((blk, _ED), lambda i: (i, 0)),
            pl.BlockSpec((blk, F), lambda i: (i, 0)),
            full((_ED, 64)),
            full((_ED, 64)),
            full((F, 64)),
            full((1, 64)),
            full((64, 32)),
            full((1, 32)),
            full((32, 16)),
            full((1, 16)),
            full((1, 16)),
            full((1, 1)),
        ],
        out_specs=pl.BlockSpec((blk, 1), lambda i: (i, 0)),
        out_shape=jax.ShapeDtypeStruct((B, 1), jnp.float32),
    )


def kernel(x, emb_a, emb_b, W1, b1, W2, b2, W3, b3, Wo, bo):
    B, C = x.shape
    F = C - 2
    V = emb_a.shape[0]
    ids_a = x[:, 0].astype(jnp.int32)
    ids_b = x[:, 1].astype(jnp.int32)
    numeric = x[:, 2:]
    pa, pb = _sc_repack(V)(emb_a.T, emb_b.T)
    covered = V // 128 * 128
    if covered < V:
        pa = pa.at[covered * _ED:].set(emb_a[covered:].reshape(-1))
        pb = pb.at[covered * _ED:].set(emb_b[covered:].reshape(-1))
    pa = pa.reshape(V, _ED)
    pb = pb.reshape(V, _ED)
    ea, eb = _sc_gather2(B, V)(ids_a, ids_b, pa, pb)
    W1T = W1.T
    out = _mlp_call(B, F, 2048)(
        ea, eb, numeric,
        W1T[:_ED], W1T[_ED:2 * _ED], W1T[2 * _ED:],
        b1.reshape(1, 64), W2.T, b2.reshape(1, 32), W3.T, b3.reshape(1, 16),
        Wo, bo.reshape(1, 1))
    return out


# speedup vs baseline: 2.8646x; 1.1289x over previous
import functools

import jax
import jax.numpy as jnp
from jax import lax
from jax.experimental import pallas as pl
from jax.experimental.pallas import tpu as pltpu
from jax.experimental.pallas import tpu_sc as plsc

_ED = 16
_IDX_CHUNK = 128
_LANES = 3200
_CW = 640


def _iota16():
    return lax.iota(jnp.int32, 16)


@functools.cache
def _sc_repack(V: int):
    info = plsc.get_sparse_core_info()
    nc, ns = info.num_cores, info.num_subcores
    nw = nc * ns
    tail_lo = (nw - 1) * _LANES
    tail_w = (V - tail_lo) // 128 * 128
    tail_cw = 384
    assert tail_w % tail_cw == 0 and _LANES % _CW == 0
    mesh = plsc.VectorSubcoreMesh(core_axis_name="c", subcore_axis_name="s")

    @functools.partial(
        pl.kernel,
        out_type=(
            jax.ShapeDtypeStruct((V // 8 * 128,), jnp.float32),
            jax.ShapeDtypeStruct((V // 8 * 128,), jnp.float32),
        ),
        mesh=mesh,
        compiler_params=pltpu.CompilerParams(needs_layout_passes=False),
        scratch_types=[
            pltpu.VMEM((16, _CW), jnp.float32),
            pltpu.VMEM((16, _CW), jnp.float32),
            pltpu.VMEM((16 * _CW,), jnp.float32),
            pltpu.VMEM((16 * _CW,), jnp.float32),
            pltpu.SemaphoreType.DMA,
            pltpu.SemaphoreType.DMA,
            pltpu.SemaphoreType.DMA,
            pltpu.SemaphoreType.DMA,
        ],
    )
    def repack(taT, tbT, pa, pb, buf0, buf1, st0, st1,
               sin0, sin1, sout0, sout1):
        wid = lax.axis_index("s") * nc + lax.axis_index("c")
        iota = _iota16()
        sbase = (iota // 8) * 128 + 16 * (iota % 8)
        bufs = (buf0, buf1)
        stages = (st0, st1)
        sins = (sin0, sin1)
        souts = (sout0, sout1)

        def pack_chunk(buf, stage, cw):
            def body(t2, _):
                base_flat = 256 * t2 + sbase
                for f in range(16):
                    vals = buf[f, pl.ds(16 * t2, 16)]
                    plsc.store_scatter(stage, [base_flat + f], vals)
                return _

            lax.fori_loop(0, cw // 16, body, 0)

        def run(chunks, cw):
            n = len(chunks)
            infl_in = [None, None]
            infl_out = [None, None]

            def start_in(i):
                src, _, lo = chunks[i]
                p = i % 2
                infl_in[p] = pltpu.async_copy(
                    src.at[:, pl.ds(pl.multiple_of(lo, 128), cw)],
                    bufs[p].at[:, pl.ds(0, cw)], sins[p])

            start_in(0)
            for i in range(n):
                p = i % 2
                infl_in[p].wait()
                if i + 1 < n:
                    start_in(i + 1)
                if infl_out[p] is not None:
                    infl_out[p].wait()
                pack_chunk(bufs[p], stages[p], cw)
                _, dst, lo = chunks[i]
                infl_out[p] = pltpu.async_copy(
                    stages[p].at[pl.ds(0, cw * 16)],
                    dst.at[pl.ds(pl.multiple_of(lo * 16, 2048), cw * 16)],
                    souts[p])
            for c in infl_out:
                if c is not None:
                    c.wait()

        @pl.when(wid < nw - 1)
        def _():
            lo = pl.multiple_of(wid * _LANES, _LANES)
            chunks = []
            for tab, dst in ((taT, pa), (tbT, pb)):
                for k in range(_LANES // _CW):
                    chunks.append((tab, dst, lo + k * _CW))
            run(chunks, _CW)

        @pl.when(wid == nw - 1)
        def _():
            chunks = []
            for tab, dst in ((taT, pa), (tbT, pb)):
                for k in range(tail_w // tail_cw):
                    chunks.append((tab, dst, tail_lo + k * tail_cw))
            run(chunks, tail_cw)

    return repack


@functools.cache
def _sc_gather2(B: int, V: int):
    info = plsc.get_sparse_core_info()
    nc, ns = info.num_cores, info.num_subcores
    nw = nc * ns
    bpw = B // nw
    n_chunks = bpw // _IDX_CHUNK
    assert bpw % _IDX_CHUNK == 0 and B % nw == 0
    mesh = plsc.VectorSubcoreMesh(core_axis_name="c", subcore_axis_name="s")

    @functools.partial(
        pl.kernel,
        out_type=(
            jax.ShapeDtypeStruct((B, 128), jnp.float32),
            jax.ShapeDtypeStruct((B, 128), jnp.float32),
        ),
        mesh=mesh,
        compiler_params=pltpu.CompilerParams(use_tc_tiling_on_sc=False),
        scratch_types=[
            pltpu.VMEM((bpw,), jnp.int32),
            pltpu.VMEM((bpw,), jnp.int32),
            pltpu.VMEM((bpw, _ED), jnp.float32),
            pltpu.VMEM((bpw, _ED), jnp.float32),
            pltpu.SemaphoreType.DMA,
            pltpu.SemaphoreType.DMA,
        ],
    )
    def gather2(ids_a_hbm, ids_b_hbm, tab_a, tab_b, out_a, out_b,
                idx_a, idx_b, rows_a, rows_b, sem_a, sem_b):
        wid = lax.axis_index("s") * nc + lax.axis_index("c")
        base = pl.multiple_of(wid * bpw, bpw)
        ca = pltpu.async_copy(ids_a_hbm.at[pl.ds(base, bpw)], idx_a, sem_a)
        cb = pltpu.async_copy(ids_b_hbm.at[pl.ds(base, bpw)], idx_b, sem_b)
        ca.wait()
        cb.wait()
        copies = []
        for j in range(n_chunks):
            copies.append(pltpu.async_copy(
                tab_a.at[idx_a.at[pl.ds(j * _IDX_CHUNK, _IDX_CHUNK)]],
                rows_a.at[pl.ds(j * _IDX_CHUNK, _IDX_CHUNK)], sem_a))
            copies.append(pltpu.async_copy(
                tab_b.at[idx_b.at[pl.ds(j * _IDX_CHUNK, _IDX_CHUNK)]],
                rows_b.at[pl.ds(j * _IDX_CHUNK, _IDX_CHUNK)], sem_b))
        for c in copies:
            c.wait()
        wa = pltpu.async_copy(rows_a, out_a.at[pl.ds(base, bpw), pl.ds(0, _ED)],
                              sem_a)
        wb = pltpu.async_copy(rows_b, out_b.at[pl.ds(base, bpw), pl.ds(0, _ED)],
                              sem_b)
        wa.wait()
        wb.wait()

    return gather2


def _mlp_body(ea, eb, num, w1a, w1b, w1n, b1, w2, b2, w3, b3, wo, bo, out):
    h = jnp.dot(ea[...][:, :_ED], w1a[...], preferred_element_type=jnp.float32)
    h = h + jnp.dot(eb[...][:, :_ED], w1b[...], preferred_element_type=jnp.float32)
    h = h + jnp.dot(num[...], w1n[...], preferred_element_type=jnp.float32)
    h = jnp.maximum(h + b1[...], 0.0)
    h = jnp.maximum(
        jnp.dot(h, w2[...], preferred_element_type=jnp.float32) + b2[...], 0.0)
    h = jnp.maximum(
        jnp.dot(h, w3[...], preferred_element_type=jnp.float32) + b3[...], 0.0)
    z = jnp.sum(h * wo[...], axis=1, keepdims=True) + bo[...]
    out[...] = 1.0 / (1.0 + jnp.exp(-z))


@functools.cache
def _mlp_call(B: int, F: int, blk: int):
    full = lambda shape: pl.BlockSpec(shape, lambda i: (0, 0))
    return pl.pallas_call(
        _mlp_body,
        grid=(B // blk,),
        in_specs=[
            pl.BlockSpec((blk, 128), lambda i: (i, 0)),
            pl.BlockSpec((blk, 128), lambda i: (i, 0)),
            pl.BlockSpec((blk, F), lambda i: (i, 0)),
            full((_ED, 64)),
            full((_ED, 64)),
            full((F, 64)),
            full((1, 64)),
            full((64, 32)),
            full((1, 32)),
            full((32, 16)),
            full((1, 16)),
            full((1, 16)),
            full((1, 1)),
        ],
        out_specs=pl.BlockSpec((blk, 1), lambda i: (i, 0)),
        out_shape=jax.ShapeDtypeStruct((B, 1), jnp.float32),
    )


def kernel(x, emb_a, emb_b, W1, b1, W2, b2, W3, b3, Wo, bo):
    B, C = x.shape
    F = C - 2
    V = emb_a.shape[0]
    ids_a = x[:, 0].astype(jnp.int32)
    ids_b = x[:, 1].astype(jnp.int32)
    numeric = x[:, 2:]
    pa, pb = _sc_repack(V)(emb_a.T, emb_b.T)
    covered = V // 128 * 128
    if covered < V:
        pa = pa.at[covered * _ED:].set(emb_a[covered:].reshape(-1))
        pb = pb.at[covered * _ED:].set(emb_b[covered:].reshape(-1))
    pa = pa.reshape(V, _ED)
    pb = pb.reshape(V, _ED)
    ea, eb = _sc_gather2(B, V)(ids_a, ids_b, pa, pb)
    W1T = W1.T
    out = _mlp_call(B, F, 2048)(
        ea, eb, numeric,
        W1T[:_ED], W1T[_ED:2 * _ED], W1T[2 * _ED:],
        b1.reshape(1, 64), W2.T, b2.reshape(1, 32), W3.T, b3.reshape(1, 16),
        Wo, bo.reshape(1, 1))
    return out


# speedup vs baseline: 2.9814x; 1.0408x over previous
import functools

import jax
import jax.numpy as jnp
from jax import lax
from jax.experimental import pallas as pl
from jax.experimental.pallas import tpu as pltpu
from jax.experimental.pallas import tpu_sc as plsc

_ED = 16
_IDX_CHUNK = 128
_LANES = 3200
_CW = 640


def _iota16():
    return lax.iota(jnp.int32, 16)


@functools.cache
def _sc_repack(V: int):
    info = plsc.get_sparse_core_info()
    nc, ns = info.num_cores, info.num_subcores
    nw = nc * ns
    tail_lo = (nw - 1) * _LANES
    tail_w = (V - tail_lo) // 128 * 128
    tail_cw = 384
    assert tail_w % tail_cw == 0 and _LANES % _CW == 0
    mesh = plsc.VectorSubcoreMesh(core_axis_name="c", subcore_axis_name="s")

    @functools.partial(
        pl.kernel,
        out_type=(
            jax.ShapeDtypeStruct((V // 8 * 128,), jnp.float32),
            jax.ShapeDtypeStruct((V // 8 * 128,), jnp.float32),
        ),
        mesh=mesh,
        compiler_params=pltpu.CompilerParams(needs_layout_passes=False),
        scratch_types=[
            pltpu.VMEM((16, _CW), jnp.float32),
            pltpu.VMEM((16, _CW), jnp.float32),
            pltpu.VMEM((16, _CW), jnp.float32),
            pltpu.VMEM((16 * _CW,), jnp.float32),
            pltpu.VMEM((16 * _CW,), jnp.float32),
            pltpu.VMEM((16 * _CW,), jnp.float32),
            pltpu.SemaphoreType.DMA,
            pltpu.SemaphoreType.DMA,
            pltpu.SemaphoreType.DMA,
            pltpu.SemaphoreType.DMA,
            pltpu.SemaphoreType.DMA,
            pltpu.SemaphoreType.DMA,
        ],
    )
    def repack(taT, tbT, pa, pb, buf0, buf1, buf2, st0, st1, st2,
               sin0, sin1, sin2, sout0, sout1, sout2):
        wid = lax.axis_index("s") * nc + lax.axis_index("c")
        iota = _iota16()
        sbase = (iota // 8) * 128 + 16 * (iota % 8)
        bufs = (buf0, buf1, buf2)
        stages = (st0, st1, st2)
        sins = (sin0, sin1, sin2)
        souts = (sout0, sout1, sout2)

        def pack_chunk(buf, stage, cw):
            def body(t2, _):
                base_flat = 256 * t2 + sbase
                for f in range(16):
                    vals = buf[f, pl.ds(16 * t2, 16)]
                    plsc.store_scatter(stage, [base_flat + f], vals)
                return _

            lax.fori_loop(0, cw // 16, body, 0)

        def run(chunks, cw):
            n = len(chunks)
            ring = 3
            infl_in = [None] * ring
            infl_out = [None] * ring

            def start_in(i):
                src, _, lo = chunks[i]
                p = i % ring
                infl_in[p] = pltpu.async_copy(
                    src.at[:, pl.ds(pl.multiple_of(lo, 128), cw)],
                    bufs[p].at[:, pl.ds(0, cw)], sins[p])

            for i in range(min(2, n)):
                start_in(i)
            for i in range(n):
                p = i % ring
                infl_in[p].wait()
                if i + 2 < n:
                    start_in(i + 2)
                if infl_out[p] is not None:
                    infl_out[p].wait()
                pack_chunk(bufs[p], stages[p], cw)
                _, dst, lo = chunks[i]
                infl_out[p] = pltpu.async_copy(
                    stages[p].at[pl.ds(0, cw * 16)],
                    dst.at[pl.ds(pl.multiple_of(lo * 16, 2048), cw * 16)],
                    souts[p])
            for c in infl_out:
                if c is not None:
                    c.wait()

        @pl.when(wid < nw - 1)
        def _():
            lo = pl.multiple_of(wid * _LANES, _LANES)
            chunks = []
            for tab, dst in ((taT, pa), (tbT, pb)):
                for k in range(_LANES // _CW):
                    chunks.append((tab, dst, lo + k * _CW))
            run(chunks, _CW)

        @pl.when(wid == nw - 1)
        def _():
            chunks = []
            for tab, dst in ((taT, pa), (tbT, pb)):
                for k in range(tail_w // tail_cw):
                    chunks.append((tab, dst, tail_lo + k * tail_cw))
            run(chunks, tail_cw)

    return repack


@functools.cache
def _sc_gather2(B: int, V: int):
    info = plsc.get_sparse_core_info()
    nc, ns = info.num_cores, info.num_subcores
    nw = nc * ns
    bpw = B // nw
    n_chunks = bpw // _IDX_CHUNK
    assert bpw % _IDX_CHUNK == 0 and B % nw == 0
    mesh = plsc.VectorSubcoreMesh(core_axis_name="c", subcore_axis_name="s")

    @functools.partial(
        pl.kernel,
        out_type=jax.ShapeDtypeStruct((B, 128), jnp.float32),
        mesh=mesh,
        compiler_params=pltpu.CompilerParams(use_tc_tiling_on_sc=False),
        scratch_types=[
            pltpu.VMEM((bpw,), jnp.int32),
            pltpu.VMEM((bpw,), jnp.int32),
            pltpu.VMEM((bpw, _ED), jnp.float32),
            pltpu.VMEM((bpw, _ED), jnp.float32),
            pltpu.SemaphoreType.DMA,
            pltpu.SemaphoreType.DMA,
        ],
    )
    def gather2(ids_a_hbm, ids_b_hbm, tab_a, tab_b, out_ab,
                idx_a, idx_b, rows_a, rows_b, sem_a, sem_b):
        wid = lax.axis_index("s") * nc + lax.axis_index("c")
        base = pl.multiple_of(wid * bpw, bpw)
        ca = pltpu.async_copy(ids_a_hbm.at[pl.ds(base, bpw)], idx_a, sem_a)
        cb = pltpu.async_copy(ids_b_hbm.at[pl.ds(base, bpw)], idx_b, sem_b)
        ca.wait()
        cb.wait()
        copies = []
        for j in range(n_chunks):
            copies.append(pltpu.async_copy(
                tab_a.at[idx_a.at[pl.ds(j * _IDX_CHUNK, _IDX_CHUNK)]],
                rows_a.at[pl.ds(j * _IDX_CHUNK, _IDX_CHUNK)], sem_a))
            copies.append(pltpu.async_copy(
                tab_b.at[idx_b.at[pl.ds(j * _IDX_CHUNK, _IDX_CHUNK)]],
                rows_b.at[pl.ds(j * _IDX_CHUNK, _IDX_CHUNK)], sem_b))
        for c in copies:
            c.wait()
        wa = pltpu.async_copy(rows_a,
                              out_ab.at[pl.ds(base, bpw), pl.ds(0, _ED)],
                              sem_a)
        wb = pltpu.async_copy(rows_b,
                              out_ab.at[pl.ds(base, bpw), pl.ds(_ED, _ED)],
                              sem_b)
        wa.wait()
        wb.wait()

    return gather2


def _mlp_body(eab, num, w1a, w1b, w1n, b1, w2, b2, w3, b3, wo, bo, out):
    e = eab[...]
    h = jnp.dot(e[:, :_ED], w1a[...], preferred_element_type=jnp.float32)
    h = h + jnp.dot(e[:, _ED:2 * _ED], w1b[...], preferred_element_type=jnp.float32)
    h = h + jnp.dot(num[...], w1n[...], preferred_element_type=jnp.float32)
    h = jnp.maximum(h + b1[...], 0.0)
    h = jnp.maximum(
        jnp.dot(h, w2[...], preferred_element_type=jnp.float32) + b2[...], 0.0)
    h = jnp.maximum(
        jnp.dot(h, w3[...], preferred_element_type=jnp.float32) + b3[...], 0.0)
    z = jnp.sum(h * wo[...], axis=1, keepdims=True) + bo[...]
    out[...] = 1.0 / (1.0 + jnp.exp(-z))


@functools.cache
def _mlp_call(B: int, F: int, blk: int):
    full = lambda shape: pl.BlockSpec(shape, lambda i: (0, 0))
    return pl.pallas_call(
        _mlp_body,
        grid=(B // blk,),
        in_specs=[
            pl.BlockSpec((blk, 128), lambda i: (i, 0)),
            pl.BlockSpec((blk, F), lambda i: (i, 0)),
            full((_ED, 64)),
            full((_ED, 64)),
            full((F, 64)),
            full((1, 64)),
            full((64, 32)),
            full((1, 32)),
            full((32, 16)),
            full((1, 16)),
            full((1, 16)),
            full((1, 1)),
        ],
        out_specs=pl.BlockSpec((blk, 1), lambda i: (i, 0)),
        out_shape=jax.ShapeDtypeStruct((B, 1), jnp.float32),
    )


def kernel(x, emb_a, emb_b, W1, b1, W2, b2, W3, b3, Wo, bo):
    B, C = x.shape
    F = C - 2
    V = emb_a.shape[0]
    ids_a = x[:, 0].astype(jnp.int32)
    ids_b = x[:, 1].astype(jnp.int32)
    numeric = x[:, 2:]
    pa, pb = _sc_repack(V)(emb_a.T, emb_b.T)
    covered = V // 128 * 128
    if covered < V:
        pa = pa.at[covered * _ED:].set(emb_a[covered:].reshape(-1))
        pb = pb.at[covered * _ED:].set(emb_b[covered:].reshape(-1))
    pa = pa.reshape(V, _ED)
    pb = pb.reshape(V, _ED)
    eab = _sc_gather2(B, V)(ids_a, ids_b, pa, pb)
    W1T = W1.T
    out = _mlp_call(B, F, 4096)(
        eab, numeric,
        W1T[:_ED], W1T[_ED:2 * _ED], W1T[2 * _ED:],
        b1.reshape(1, 64), W2.T, b2.reshape(1, 32), W3.T, b3.reshape(1, 16),
        Wo, bo.reshape(1, 1))
    return out
